# baseline (speedup 1.0000x reference)
import jax
import jax.numpy as jnp
from jax.experimental import pallas as pl
from jax.experimental.pallas import tpu as pltpu

_NPAIR = 16
_H = 32


def _build_cols(x):
    b = x.shape[0]
    h = x.shape[1]
    ho = h // 2
    p = jax.lax.conv_general_dilated_patches(
        x[:, None, :, :], (3, 3), (2, 2), ((1, 1), (1, 1)))
    return p.reshape(b, 9, ho * ho)


def _s2_tap(pv, n_img, hp, ho, c, kh, kw):
    ah, ph = kh // 2, kh % 2
    aw, pw = kw // 2, kw % 2
    q = pv.reshape(n_img, hp // 2, 2, hp, c)[:, ah:ah + ho, ph:ph + 1, :, :]
    q = q.reshape(n_img, ho, hp, c)
    s = q.reshape(n_img, ho, hp // 2, 2, c)[:, :, aw:aw + ho, pw:pw + 1, :]
    return s.reshape(n_img * ho * ho, c)


def _encoder_kernel(xa_ref, xb_ref, sc_ref,
                    w10_ref, corr0_ref, wd0_ref, b2be0_ref, w20_ref,
                    w11_ref, wd1_ref, w21_ref,
                    w12_ref, wd2_ref, w22_ref, fcw_ref,
                    o_ref,
                    p0, p2, p4, im0, im1, im2, im3, im4):
    f32 = jnp.float32
    n_pair = xa_ref.shape[0]
    n_img = 2 * n_pair

    for pref, hp, cc in ((p0, 18, 16), (p2, 10, 32), (p4, 6, 128)):
        pref[:, 0:1, :, :] = jnp.zeros((n_img, 1, hp, cc), f32)
        pref[:, hp - 1:hp, :, :] = jnp.zeros((n_img, 1, hp, cc), f32)
        pref[:, :, 0:1, :] = jnp.zeros((n_img, hp, 1, cc), f32)
        pref[:, :, hp - 1:hp, :] = jnp.zeros((n_img, hp, 1, cc), f32)

    xc = jnp.concatenate([xa_ref[...], xb_ref[...]], axis=0)
    xc = jnp.transpose(xc, (0, 2, 1)).reshape(n_img * 256, 9)
    h = jnp.dot(xc, w10_ref[...], preferred_element_type=f32)
    h = h.reshape(n_img, 256, 16) + corr0_ref[...]
    h = jnp.maximum(h, 0.0).reshape(n_img * 256, 16)
    sc0 = xc[:, 4:5] * wd0_ref[...]

    p0[:, 1:17, 1:17, :] = (h + sc_ref[0]).reshape(n_img, 16, 16, 16)
    for kh in range(3):
        for kw in range(3):
            k = 3 * kh + kw
            im0[:, 16 * k:16 * k + 16] = (
                p0[:, kh:kh + 16, kw:kw + 16, :].reshape(n_img * 256, 16))
    o = jnp.dot(im0[...], w20_ref[...], preferred_element_type=f32)
    o = jnp.maximum(o * sc_ref[1] + b2be0_ref[...] + sc0, 0.0)

    p0[:, 1:17, 1:17, :] = (o + sc_ref[2]).reshape(n_img, 16, 16, 16)
    pv = p0[...]
    for kh in range(3):
        for kw in range(3):
            k = 3 * kh + kw
            im1[:, 16 * k:16 * k + 16] = _s2_tap(pv, n_img, 18, 8, 16, kh, kw)
    h = jnp.dot(im1[...], w11_ref[...], preferred_element_type=f32)
    h = jnp.maximum(h + sc_ref[3], 0.0)
    xs = im1[:, 64:80]
    sc1 = jnp.dot(xs, wd1_ref[...], preferred_element_type=f32)

    p2[:, 1:9, 1:9, :] = (h + sc_ref[4]).reshape(n_img, 8, 8, 32)
    for kh in range(3):
        for kw in range(3):
            k = 3 * kh + kw
            im2[:, 32 * k:32 * k + 32] = (
                p2[:, kh:kh + 8, kw:kw + 8, :].reshape(n_img * 64, 32))
    o = jnp.dot(im2[...], w21_ref[...], preferred_element_type=f32)
    o = jnp.maximum(o * sc_ref[5] + sc_ref[6] + sc1, 0.0)

    p2[:, 1:9, 1:9, :] = (o + sc_ref[7]).reshape(n_img, 8, 8, 32)
    pv = p2[...]
    for kh in range(3):
        for kw in range(3):
            k = 3 * kh + kw
            im3[:, 32 * k:32 * k + 32] = _s2_tap(pv, n_img, 10, 4, 32, kh, kw)
    h = jnp.dot(im3[...], w12_ref[...], preferred_element_type=f32)
    h = jnp.maximum(h + sc_ref[8], 0.0)
    xs = im3[:, 128:160]
    sc2 = jnp.dot(xs, wd2_ref[...], preferred_element_type=f32)

    p4[:, 1:5, 1:5, :] = (h + sc_ref[9]).reshape(n_img, 4, 4, 128)
    for kh in range(3):
        for kw in range(3):
            k = 3 * kh + kw
            im4[:, 128 * k:128 * k + 128] = (
                p4[:, kh:kh + 4, kw:kw + 4, :].reshape(n_img * 16, 128))
    o = jnp.dot(im4[...], w22_ref[...], preferred_element_type=f32)
    o = jnp.maximum(o * sc_ref[10] + sc_ref[11] + sc2, 0.0)

    z = o.reshape(n_img, 16, 128).sum(axis=1) * (1.0 / 16.0)
    d = z[:n_pair] - z[n_pair:]
    out = jnp.sum(d * d * fcw_ref[...], axis=1, keepdims=True) + sc_ref[12]
    o_ref[...] = out.astype(o_ref.dtype)


@jax.jit
def kernel(x1, x2, u0_w1, u0_w2, u0_wd, u0_b1a, u0_b1b, u0_b2a, u0_b2b, u0_scale,
           u1_w1, u1_w2, u1_wd, u1_b1a, u1_b1b, u1_b2a, u1_b2b, u1_scale,
           u2_w1, u2_w2, u2_wd, u2_b1a, u2_b1b, u2_b2a, u2_b2b, u2_scale,
           fc_w, fc_b):
    f32 = jnp.float32
    b = x1.shape[0]
    n_pair = _NPAIR
    n_img = 2 * n_pair
    grid = b // n_pair

    xc1 = _build_cols(x1.reshape(b, _H, _H).astype(f32))
    xc2 = _build_cols(x2.reshape(b, _H, _H).astype(f32))

    w10 = u0_w1.reshape(9, 16).astype(f32)
    w20 = u0_w2.reshape(144, 16).astype(f32)
    w11 = u1_w1.reshape(144, 32).astype(f32)
    w21 = u1_w2.reshape(288, 32).astype(f32)
    w12 = u2_w1.reshape(288, 128).astype(f32)
    w22 = u2_w2.reshape(1152, 128).astype(f32)
    wd0 = u0_wd.reshape(1, 16).astype(f32)
    wd1 = u1_wd.astype(f32)
    wd2 = u2_wd.astype(f32)
    mask_cols = _build_cols(jnp.ones((1, _H, _H), f32))[0].T
    corr0 = u0_b1a * jnp.dot(mask_cols, w10) + u0_b1b
    b2be0 = (u0_b2b + u0_b1a * wd0).reshape(1, 16)
    fcw = fc_w.reshape(1, 128).astype(f32)

    scalars = jnp.stack([u0_b2a, u0_scale,
                         u1_b1a, u1_b1b, u1_b2a, u1_scale, u1_b2b,
                         u2_b1a, u2_b1b, u2_b2a, u2_scale, u2_b2b,
                         fc_b.reshape(())]).astype(f32)

    full = lambda a: pl.BlockSpec(a.shape, lambda i: (0,) * a.ndim)
    in_specs = [
        pl.BlockSpec((n_pair, 9, 256), lambda i: (i, 0, 0)),
        pl.BlockSpec((n_pair, 9, 256), lambda i: (i, 0, 0)),
        pl.BlockSpec(memory_space=pltpu.MemorySpace.SMEM),
        full(w10), full(corr0), full(wd0), full(b2be0), full(w20),
        full(w11), full(wd1), full(w21),
        full(w12), full(wd2), full(w22), full(fcw),
    ]
    scratch = [
        pltpu.VMEM((n_img, 18, 18, 16), f32),
        pltpu.VMEM((n_img, 10, 10, 32), f32),
        pltpu.VMEM((n_img, 6, 6, 128), f32),
        pltpu.VMEM((n_img * 256, 144), f32),
        pltpu.VMEM((n_img * 64, 144), f32),
        pltpu.VMEM((n_img * 64, 288), f32),
        pltpu.VMEM((n_img * 16, 288), f32),
        pltpu.VMEM((n_img * 16, 1152), f32),
    ]
    out = pl.pallas_call(
        _encoder_kernel,
        out_shape=jax.ShapeDtypeStruct((b, 1), f32),
        grid=(grid,),
        in_specs=in_specs,
        out_specs=pl.BlockSpec((n_pair, 1), lambda i: (i, 0)),
        scratch_shapes=scratch,
        compiler_params=pltpu.CompilerParams(
            dimension_semantics=("parallel",)),
    )(xc1, xc2, scalars, w10, corr0, wd0, b2be0, w20,
      w11, wd1, w21, w12, wd2, w22, fcw)
    return out[:, 0]
